# Initial kernel scaffold; baseline (speedup 1.0000x reference)
import jax
import jax.numpy as jnp
from jax.experimental import pallas as pl


def kernel(x, W_n2h, b_n2h, W_h1, b_h1, W_h2, b_h2, W_h2n, b_h2n, W_u1, b_u1, W_u2, b_u2, gamma, beta, hyperedge_index):
    raise NotImplementedError("write your pallas kernel here")


# speedup vs baseline: 8.1359x; 8.1359x over previous
import functools

import jax
import jax.numpy as jnp
from jax import lax
from jax.experimental import pallas as pl
from jax.experimental.pallas import tpu as pltpu
from jax.experimental.pallas import tpu_sc as plsc

NC = 2
NS = 16
NW = NC * NS
LANES = 16


def _tc_node_transform(x, W, b):
    n, d_in = x.shape
    hd = W.shape[1]
    blk = 400
    assert n % blk == 0
    grid = n // blk

    def body(x_ref, w_ref, b_ref, o_ref):
        o_ref[...] = (
            jnp.dot(x_ref[...], w_ref[...], preferred_element_type=jnp.float32)
            + b_ref[...]
        )

    return pl.pallas_call(
        body,
        grid=(grid,),
        in_specs=[
            pl.BlockSpec((blk, d_in), lambda i: (i, 0)),
            pl.BlockSpec((d_in, hd), lambda i: (0, 0)),
            pl.BlockSpec((1, hd), lambda i: (0, 0)),
        ],
        out_specs=pl.BlockSpec((blk, hd), lambda i: (i, 0)),
        out_shape=jax.ShapeDtypeStruct((n, hd), jnp.float32),
    )(x, W, b.reshape(1, hd))


def _tc_hedge_mlp(hs, W1s, b1, W2, b2, n_real):
    ph, hd = hs.shape
    blk = 512
    assert ph % blk == 0
    grid = ph // blk

    def body(hs_ref, w1_ref, b1_ref, w2_ref, b2_ref, o_ref):
        i = pl.program_id(0)
        t = jnp.dot(hs_ref[...], w1_ref[...], preferred_element_type=jnp.float32)
        t = jnp.maximum(t + b1_ref[...], 0.0)
        o = jnp.dot(t, w2_ref[...], preferred_element_type=jnp.float32) + b2_ref[...]
        rows = i * blk + lax.broadcasted_iota(jnp.int32, (blk, 1), 0)
        o_ref[...] = jnp.where(rows < n_real, o, 0.0)

    return pl.pallas_call(
        body,
        grid=(grid,),
        in_specs=[
            pl.BlockSpec((blk, hd), lambda i: (i, 0)),
            pl.BlockSpec((hd, hd), lambda i: (0, 0)),
            pl.BlockSpec((1, hd), lambda i: (0, 0)),
            pl.BlockSpec((hd, hd), lambda i: (0, 0)),
            pl.BlockSpec((1, hd), lambda i: (0, 0)),
        ],
        out_specs=pl.BlockSpec((blk, hd), lambda i: (i, 0)),
        out_shape=jax.ShapeDtypeStruct((ph, hd), jnp.float32),
    )(hs, W1s, b1.reshape(1, hd), W2, b2.reshape(1, hd))


def _sc_gather_sum(x_t, idx_g, ph, hd, card):
    hpt = ph // NW
    shot_h = 32
    shots = hpt // shot_h
    idx_rows_shot = shot_h * card // 128
    rows_shot = shot_h * card

    mesh = plsc.VectorSubcoreMesh(
        core_axis_name="c", subcore_axis_name="s",
        num_cores=NC, num_subcores=NS)

    def body(xt_hbm, idx_hbm, out_hbm, idx_v, rows_v, res_v, sem0, sem1):
        cid = lax.axis_index("c")
        sid = lax.axis_index("s")
        wid = sid * NC + cid
        sems = (sem0, sem1)

        def fire(s, b):
            row0 = wid * (shots * idx_rows_shot) + s * idx_rows_shot
            pltpu.sync_copy(idx_hbm.at[pl.ds(row0, idx_rows_shot)], idx_v.at[b])
            hs = []
            for j in range(idx_rows_shot):
                hs.append(pltpu.async_copy(
                    xt_hbm.at[idx_v.at[b, j]],
                    rows_v.at[b, pl.ds(j * 128, 128)],
                    sems[b]))
            return hs

        def reduce_shot(s, b):
            def hbody(h, _):
                base = h * card
                a0 = rows_v[b, base, pl.ds(0, LANES)]
                a1 = rows_v[b, base, pl.ds(LANES, LANES)]
                for c in range(1, card):
                    a0 = a0 + rows_v[b, base + c, pl.ds(0, LANES)]
                    a1 = a1 + rows_v[b, base + c, pl.ds(LANES, LANES)]
                res_v[s * shot_h + h, pl.ds(0, LANES)] = a0
                res_v[s * shot_h + h, pl.ds(LANES, LANES)] = a1
                return 0
            lax.fori_loop(0, shot_h, hbody, 0)

        pending = {0: fire(0, 0)}
        for s in range(shots):
            b = s % 2
            if s + 1 < shots:
                pending[s + 1] = fire(s + 1, (s + 1) % 2)
            for h in pending.pop(s):
                h.wait()
            reduce_shot(s, b)
        pltpu.sync_copy(res_v, out_hbm.at[pl.ds(wid * hpt, hpt)])

    call = pl.kernel(
        body,
        out_type=jax.ShapeDtypeStruct((ph, hd), jnp.float32),
        mesh=mesh,
        scratch_types=[
            pltpu.VMEM((2, idx_rows_shot, 128), jnp.int32),
            pltpu.VMEM((2, rows_shot, hd), jnp.float32),
            pltpu.VMEM((hpt, hd), jnp.float32),
            pltpu.SemaphoreType.DMA,
            pltpu.SemaphoreType.DMA,
        ],
        compiler_params=pltpu.CompilerParams(use_tc_tiling_on_sc=False, needs_layout_passes=False),
    )
    return call(x_t, idx_g)


def _sc_scatter(hedge_feat, idx_s, n_nodes, ph, hd, card, n_real):
    hpt = ph // NW
    hchunks = hpt // 64
    nk = card * hchunks
    rows_pt = n_nodes // NS
    zrows = hpt

    mesh = plsc.VectorSubcoreMesh(
        core_axis_name="c", subcore_axis_name="s",
        num_cores=NC, num_subcores=NS)

    def body(feat_hbm, idx_hbm, acc_hbm, cnt_hbm,
             feat_v, idx_v, cnt_v, acc_sp, sem):
        zero16 = jnp.zeros((LANES,), jnp.float32)
        cid = lax.axis_index("c")
        sid = lax.axis_index("s")
        wid = sid * NC + cid

        def zrow(r, _):
            feat_v[r, pl.ds(0, LANES)] = zero16
            feat_v[r, pl.ds(LANES, LANES)] = zero16
            return 0
        lax.fori_loop(0, zrows, zrow, 0)
        base = sid * rows_pt
        pltpu.sync_copy(feat_v, acc_sp.at[pl.ds(base, zrows)])
        pltpu.sync_copy(feat_v.at[pl.ds(0, rows_pt - zrows)],
                        acc_sp.at[pl.ds(base + zrows, rows_pt - zrows)])

        def zcnt(i, _):
            cnt_v[pl.ds(i * LANES, LANES)] = zero16
            return 0
        lax.fori_loop(0, n_nodes // LANES, zcnt, 0)

        plsc.subcore_barrier()

        pltpu.sync_copy(feat_hbm.at[pl.ds(wid * hpt, hpt)], feat_v)
        pltpu.sync_copy(idx_hbm.at[wid], idx_v)

        iota16 = lax.iota(jnp.int32, LANES)

        def cbody(k, _):
            hchunk = lax.rem(k, hchunks)
            for j in range(64 // LANES):
                iv = idx_v[k, pl.ds(j * LANES, LANES)]
                rows = wid * hpt + hchunk * 64 + j * LANES + iota16
                ones = jnp.where(rows < n_real, 1.0, 0.0)
                plsc.addupdate_scatter(cnt_v, [iv], ones)
            return 0
        lax.fori_loop(0, nk, cbody, 0)

        handles = []
        for k in range(nk):
            hc = k % hchunks
            handles.append(pltpu.async_copy(
                feat_v.at[pl.ds(hc * 64, 64)],
                acc_sp.at[idx_v.at[k]],
                sem, add=True))
            if len(handles) == 8:
                for h in handles:
                    h.wait()
                handles = []
        for h in handles:
            h.wait()

        plsc.subcore_barrier()

        pltpu.sync_copy(acc_sp.at[pl.ds(base, rows_pt)],
                        acc_hbm.at[cid, pl.ds(base, rows_pt)])
        pltpu.sync_copy(cnt_v, cnt_hbm.at[wid])

    call = pl.kernel(
        body,
        out_type=(
            jax.ShapeDtypeStruct((NC, n_nodes, hd), jnp.float32),
            jax.ShapeDtypeStruct((NW, n_nodes), jnp.float32),
        ),
        mesh=mesh,
        scratch_types=[
            pltpu.VMEM((hpt, hd), jnp.float32),
            pltpu.VMEM((nk, 64), jnp.int32),
            pltpu.VMEM((n_nodes,), jnp.float32),
            pltpu.VMEM_SHARED((n_nodes, hd), jnp.float32),
            pltpu.SemaphoreType.DMA,
        ],
        compiler_params=pltpu.CompilerParams(use_tc_tiling_on_sc=False, needs_layout_passes=False),
    )
    return call(hedge_feat, idx_s)


def _tc_final(x, acc, cnt, W_h2n, b_h2n, W_u1, b_u1, W_u2, b_u2, gamma, beta):
    n, d_in = x.shape
    hd = acc.shape[-1]
    d_out = W_h2n.shape[1]
    blk = 400
    assert n % blk == 0
    grid = n // blk

    def body(x_ref, acc_ref, cnt_ref, wh_ref, bh_ref, w1_ref, b1_ref,
             w2_ref, b2_ref, g_ref, be_ref, o_ref):
        cnt = jnp.maximum(jnp.sum(cnt_ref[...], axis=1), 1.0)
        nf = (acc_ref[0] + acc_ref[1]) / cnt[:, None]
        nf = jnp.dot(nf, wh_ref[...], preferred_element_type=jnp.float32) + bh_ref[...]
        w1 = w1_ref[...]
        u = (jnp.dot(x_ref[...], w1[:d_in], preferred_element_type=jnp.float32)
             + jnp.dot(nf, w1[d_in:], preferred_element_type=jnp.float32)
             + b1_ref[...])
        u = jnp.maximum(u, 0.0)
        o = jnp.dot(u, w2_ref[...], preferred_element_type=jnp.float32) + b2_ref[...]
        mu = jnp.mean(o, axis=-1, keepdims=True)
        var = jnp.mean((o - mu) ** 2, axis=-1, keepdims=True)
        o_ref[...] = (o - mu) * lax.rsqrt(var + 1e-5) * g_ref[...] + be_ref[...]

    return pl.pallas_call(
        body,
        grid=(grid,),
        in_specs=[
            pl.BlockSpec((blk, d_in), lambda i: (i, 0)),
            pl.BlockSpec((NC, blk, hd), lambda i: (0, i, 0)),
            pl.BlockSpec((blk, NW), lambda i: (i, 0)),
            pl.BlockSpec((hd, d_out), lambda i: (0, 0)),
            pl.BlockSpec((1, d_out), lambda i: (0, 0)),
            pl.BlockSpec((d_in + d_out, d_out), lambda i: (0, 0)),
            pl.BlockSpec((1, d_out), lambda i: (0, 0)),
            pl.BlockSpec((d_out, d_out), lambda i: (0, 0)),
            pl.BlockSpec((1, d_out), lambda i: (0, 0)),
            pl.BlockSpec((1, d_out), lambda i: (0, 0)),
            pl.BlockSpec((1, d_out), lambda i: (0, 0)),
        ],
        out_specs=pl.BlockSpec((blk, d_out), lambda i: (i, 0)),
        out_shape=jax.ShapeDtypeStruct((n, d_out), jnp.float32),
    )(x, acc, cnt, W_h2n, b_h2n.reshape(1, d_out), W_u1,
      b_u1.reshape(1, d_out), W_u2, b_u2.reshape(1, d_out),
      gamma.reshape(1, d_out), beta.reshape(1, d_out))


def kernel(x, W_n2h, b_n2h, W_h1, b_h1, W_h2, b_h2, W_h2n, b_h2n,
           W_u1, b_u1, W_u2, b_u2, gamma, beta, hyperedge_index):
    n_nodes = x.shape[0]
    h_real, card = hyperedge_index.shape
    hd = W_n2h.shape[1]

    ph = ((h_real + NW * 320 - 1) // (NW * 320)) * (NW * 320)
    hpt = ph // NW

    idx_pad = jnp.zeros((ph, card), jnp.int32).at[:h_real].set(hyperedge_index)
    idx_g = idx_pad.reshape(ph * card // 128, 128)
    idx_s = (idx_pad.reshape(NW, hpt, card)
             .transpose(0, 2, 1)
             .reshape(NW, card * (hpt // 64), 64))

    x_t = _tc_node_transform(x, W_n2h, b_n2h)
    hedge_sums = _sc_gather_sum(x_t, idx_g, ph, hd, card)
    hedge_feat = _tc_hedge_mlp(hedge_sums, W_h1 / card, b_h1, W_h2, b_h2, h_real)
    acc, cnt = _sc_scatter(hedge_feat, idx_s, n_nodes, ph, hd, card, h_real)
    return _tc_final(x, acc, cnt.T, W_h2n, b_h2n, W_u1, b_u1, W_u2, b_u2,
                     gamma, beta)


# speedup vs baseline: 11.8948x; 1.4620x over previous
import functools

import jax
import jax.numpy as jnp
from jax import lax
from jax.experimental import pallas as pl
from jax.experimental.pallas import tpu as pltpu
from jax.experimental.pallas import tpu_sc as plsc

NC = 2
NS = 16
NW = NC * NS
LANES = 16


def _tc_node_transform(x, W, b):
    n, d_in = x.shape
    hd = W.shape[1]
    blk = 400
    assert n % blk == 0
    grid = n // blk

    def body(x_ref, w_ref, b_ref, o_ref):
        o_ref[...] = (
            jnp.dot(x_ref[...], w_ref[...], preferred_element_type=jnp.float32)
            + b_ref[...]
        )

    return pl.pallas_call(
        body,
        grid=(grid,),
        in_specs=[
            pl.BlockSpec((blk, d_in), lambda i: (i, 0)),
            pl.BlockSpec((d_in, hd), lambda i: (0, 0)),
            pl.BlockSpec((1, hd), lambda i: (0, 0)),
        ],
        out_specs=pl.BlockSpec((blk, hd), lambda i: (i, 0)),
        out_shape=jax.ShapeDtypeStruct((n, hd), jnp.float32),
    )(x, W, b.reshape(1, hd))


def _tc_hedge_mlp(hs, W1s, b1, W2, b2, n_real):
    ph, hd = hs.shape
    blk = 512
    assert ph % blk == 0
    grid = ph // blk

    def body(hs_ref, w1_ref, b1_ref, w2_ref, b2_ref, o_ref):
        i = pl.program_id(0)
        t = jnp.dot(hs_ref[...], w1_ref[...], preferred_element_type=jnp.float32)
        t = jnp.maximum(t + b1_ref[...], 0.0)
        o = jnp.dot(t, w2_ref[...], preferred_element_type=jnp.float32) + b2_ref[...]
        rows = i * blk + lax.broadcasted_iota(jnp.int32, (blk, 1), 0)
        o_ref[...] = jnp.where(rows < n_real, o, 0.0)

    return pl.pallas_call(
        body,
        grid=(grid,),
        in_specs=[
            pl.BlockSpec((blk, hd), lambda i: (i, 0)),
            pl.BlockSpec((hd, hd), lambda i: (0, 0)),
            pl.BlockSpec((1, hd), lambda i: (0, 0)),
            pl.BlockSpec((hd, hd), lambda i: (0, 0)),
            pl.BlockSpec((1, hd), lambda i: (0, 0)),
        ],
        out_specs=pl.BlockSpec((blk, hd), lambda i: (i, 0)),
        out_shape=jax.ShapeDtypeStruct((ph, hd), jnp.float32),
    )(hs, W1s, b1.reshape(1, hd), W2, b2.reshape(1, hd))


def _sc_gather_sum(x_t, idx_g, ph, hd, card):
    hpt = ph // NW
    shot_h = 32
    shots = hpt // shot_h
    idx_rows_shot = shot_h * card // 128
    rows_shot = shot_h * card

    mesh = plsc.VectorSubcoreMesh(
        core_axis_name="c", subcore_axis_name="s",
        num_cores=NC, num_subcores=NS)

    n_nodes = x_t.shape[0]
    stage_rows = n_nodes // NS

    def body(xt_hbm, idx_hbm, out_hbm, idx_v, rows_v, res_v, xt_sp, sem0, sem1):
        cid = lax.axis_index("c")
        sid = lax.axis_index("s")
        wid = sid * NC + cid
        sems = (sem0, sem1)

        pltpu.sync_copy(xt_hbm.at[pl.ds(sid * stage_rows, stage_rows)],
                        xt_sp.at[pl.ds(sid * stage_rows, stage_rows)])
        plsc.subcore_barrier()

        def fire(s, b):
            row0 = wid * (shots * idx_rows_shot) + s * idx_rows_shot
            pltpu.sync_copy(idx_hbm.at[pl.ds(row0, idx_rows_shot)], idx_v.at[b])
            hs = []
            for j in range(idx_rows_shot):
                hs.append(pltpu.async_copy(
                    xt_sp.at[idx_v.at[b, j]],
                    rows_v.at[b, pl.ds(j * 128, 128)],
                    sems[b]))
            return hs

        def reduce_shot(s, b):
            def hbody(h, _):
                base = h * card
                a0 = rows_v[b, base, pl.ds(0, LANES)]
                a1 = rows_v[b, base, pl.ds(LANES, LANES)]
                for c in range(1, card):
                    a0 = a0 + rows_v[b, base + c, pl.ds(0, LANES)]
                    a1 = a1 + rows_v[b, base + c, pl.ds(LANES, LANES)]
                res_v[s * shot_h + h, pl.ds(0, LANES)] = a0
                res_v[s * shot_h + h, pl.ds(LANES, LANES)] = a1
                return 0
            lax.fori_loop(0, shot_h, hbody, 0)

        pending = {0: fire(0, 0)}
        for s in range(shots):
            b = s % 2
            if s + 1 < shots:
                pending[s + 1] = fire(s + 1, (s + 1) % 2)
            for h in pending.pop(s):
                h.wait()
            reduce_shot(s, b)
        pltpu.sync_copy(res_v, out_hbm.at[pl.ds(wid * hpt, hpt)])

    call = pl.kernel(
        body,
        out_type=jax.ShapeDtypeStruct((ph, hd), jnp.float32),
        mesh=mesh,
        scratch_types=[
            pltpu.VMEM((2, idx_rows_shot, 128), jnp.int32),
            pltpu.VMEM((2, rows_shot, hd), jnp.float32),
            pltpu.VMEM((hpt, hd), jnp.float32),
            pltpu.VMEM_SHARED((x_t.shape[0], hd), jnp.float32),
            pltpu.SemaphoreType.DMA,
            pltpu.SemaphoreType.DMA,
        ],
        compiler_params=pltpu.CompilerParams(use_tc_tiling_on_sc=False, needs_layout_passes=False),
    )
    return call(x_t, idx_g)


def _sc_scatter(hedge_feat, idx_s, n_nodes, ph, hd, card, n_real):
    hpt = ph // NW
    hchunks = hpt // 64
    nk = card * hchunks
    rows_pt = n_nodes // NS
    zrows = hpt

    mesh = plsc.VectorSubcoreMesh(
        core_axis_name="c", subcore_axis_name="s",
        num_cores=NC, num_subcores=NS)

    def body(feat_hbm, idx_hbm, acc_hbm, cnt_hbm,
             feat_v, idx_v, cnt_v, acc_sp, sem):
        zero16 = jnp.zeros((LANES,), jnp.float32)
        cid = lax.axis_index("c")
        sid = lax.axis_index("s")
        wid = sid * NC + cid

        def zrow(r, _):
            feat_v[r, pl.ds(0, LANES)] = zero16
            feat_v[r, pl.ds(LANES, LANES)] = zero16
            return 0
        lax.fori_loop(0, zrows, zrow, 0)
        base = sid * rows_pt
        pltpu.sync_copy(feat_v, acc_sp.at[pl.ds(base, zrows)])
        pltpu.sync_copy(feat_v.at[pl.ds(0, rows_pt - zrows)],
                        acc_sp.at[pl.ds(base + zrows, rows_pt - zrows)])

        def zcnt(i, _):
            cnt_v[pl.ds(i * LANES, LANES)] = zero16
            return 0
        lax.fori_loop(0, n_nodes // LANES, zcnt, 0)

        plsc.subcore_barrier()

        pltpu.sync_copy(feat_hbm.at[pl.ds(wid * hpt, hpt)], feat_v)
        pltpu.sync_copy(idx_hbm.at[wid], idx_v)

        iota16 = lax.iota(jnp.int32, LANES)

        def cbody(k, _):
            hchunk = lax.rem(k, hchunks)
            for j in range(64 // LANES):
                iv = idx_v[k, pl.ds(j * LANES, LANES)]
                rows = wid * hpt + hchunk * 64 + j * LANES + iota16
                ones = jnp.where(rows < n_real, 1.0, 0.0)
                plsc.addupdate_scatter(cnt_v, [iv], ones)
            return 0
        lax.fori_loop(0, nk, cbody, 0)

        handles = []
        for k in range(nk):
            hc = k % hchunks
            handles.append(pltpu.async_copy(
                feat_v.at[pl.ds(hc * 64, 64)],
                acc_sp.at[idx_v.at[k]],
                sem, add=True))
            if len(handles) == 8:
                for h in handles:
                    h.wait()
                handles = []
        for h in handles:
            h.wait()

        plsc.subcore_barrier()

        pltpu.sync_copy(acc_sp.at[pl.ds(base, rows_pt)],
                        acc_hbm.at[cid, pl.ds(base, rows_pt)])
        pltpu.sync_copy(cnt_v, cnt_hbm.at[wid])

    call = pl.kernel(
        body,
        out_type=(
            jax.ShapeDtypeStruct((NC, n_nodes, hd), jnp.float32),
            jax.ShapeDtypeStruct((NW, n_nodes), jnp.float32),
        ),
        mesh=mesh,
        scratch_types=[
            pltpu.VMEM((hpt, hd), jnp.float32),
            pltpu.VMEM((nk, 64), jnp.int32),
            pltpu.VMEM((n_nodes,), jnp.float32),
            pltpu.VMEM_SHARED((n_nodes, hd), jnp.float32),
            pltpu.SemaphoreType.DMA,
        ],
        compiler_params=pltpu.CompilerParams(use_tc_tiling_on_sc=False, needs_layout_passes=False),
    )
    return call(hedge_feat, idx_s)


def _tc_final(x, acc, cnt, W_h2n, b_h2n, W_u1, b_u1, W_u2, b_u2, gamma, beta):
    n, d_in = x.shape
    hd = acc.shape[-1]
    d_out = W_h2n.shape[1]
    blk = 400
    assert n % blk == 0
    grid = n // blk

    def body(x_ref, acc_ref, cnt_ref, wh_ref, bh_ref, w1_ref, b1_ref,
             w2_ref, b2_ref, g_ref, be_ref, o_ref):
        cnt = jnp.maximum(jnp.sum(cnt_ref[...], axis=1), 1.0)
        nf = (acc_ref[0] + acc_ref[1]) / cnt[:, None]
        nf = jnp.dot(nf, wh_ref[...], preferred_element_type=jnp.float32) + bh_ref[...]
        w1 = w1_ref[...]
        u = (jnp.dot(x_ref[...], w1[:d_in], preferred_element_type=jnp.float32)
             + jnp.dot(nf, w1[d_in:], preferred_element_type=jnp.float32)
             + b1_ref[...])
        u = jnp.maximum(u, 0.0)
        o = jnp.dot(u, w2_ref[...], preferred_element_type=jnp.float32) + b2_ref[...]
        mu = jnp.mean(o, axis=-1, keepdims=True)
        var = jnp.mean((o - mu) ** 2, axis=-1, keepdims=True)
        o_ref[...] = (o - mu) * lax.rsqrt(var + 1e-5) * g_ref[...] + be_ref[...]

    return pl.pallas_call(
        body,
        grid=(grid,),
        in_specs=[
            pl.BlockSpec((blk, d_in), lambda i: (i, 0)),
            pl.BlockSpec((NC, blk, hd), lambda i: (0, i, 0)),
            pl.BlockSpec((blk, NW), lambda i: (i, 0)),
            pl.BlockSpec((hd, d_out), lambda i: (0, 0)),
            pl.BlockSpec((1, d_out), lambda i: (0, 0)),
            pl.BlockSpec((d_in + d_out, d_out), lambda i: (0, 0)),
            pl.BlockSpec((1, d_out), lambda i: (0, 0)),
            pl.BlockSpec((d_out, d_out), lambda i: (0, 0)),
            pl.BlockSpec((1, d_out), lambda i: (0, 0)),
            pl.BlockSpec((1, d_out), lambda i: (0, 0)),
            pl.BlockSpec((1, d_out), lambda i: (0, 0)),
        ],
        out_specs=pl.BlockSpec((blk, d_out), lambda i: (i, 0)),
        out_shape=jax.ShapeDtypeStruct((n, d_out), jnp.float32),
    )(x, acc, cnt, W_h2n, b_h2n.reshape(1, d_out), W_u1,
      b_u1.reshape(1, d_out), W_u2, b_u2.reshape(1, d_out),
      gamma.reshape(1, d_out), beta.reshape(1, d_out))


def kernel(x, W_n2h, b_n2h, W_h1, b_h1, W_h2, b_h2, W_h2n, b_h2n,
           W_u1, b_u1, W_u2, b_u2, gamma, beta, hyperedge_index):
    n_nodes = x.shape[0]
    h_real, card = hyperedge_index.shape
    hd = W_n2h.shape[1]

    ph = ((h_real + NW * 320 - 1) // (NW * 320)) * (NW * 320)
    hpt = ph // NW

    idx_pad = jnp.zeros((ph, card), jnp.int32).at[:h_real].set(hyperedge_index)
    idx_g = idx_pad.reshape(ph * card // 128, 128)
    idx_s = (idx_pad.reshape(NW, hpt, card)
             .transpose(0, 2, 1)
             .reshape(NW, card * (hpt // 64), 64))

    x_t = _tc_node_transform(x, W_n2h, b_n2h)
    hedge_sums = _sc_gather_sum(x_t, idx_g, ph, hd, card)
    hedge_feat = _tc_hedge_mlp(hedge_sums, W_h1 / card, b_h1, W_h2, b_h2, h_real)
    acc, cnt = _sc_scatter(hedge_feat, idx_s, n_nodes, ph, hd, card, h_real)
    return _tc_final(x, acc, cnt.T, W_h2n, b_h2n, W_u1, b_u1, W_u2, b_u2,
                     gamma, beta)


# speedup vs baseline: 14.1462x; 1.1893x over previous
import functools

import jax
import jax.numpy as jnp
from jax import lax
from jax.experimental import pallas as pl
from jax.experimental.pallas import tpu as pltpu
from jax.experimental.pallas import tpu_sc as plsc

NC = 2
NS = 16
NW = NC * NS
LANES = 16


def _tc_node_transform(x, W, b):
    n, d_in = x.shape
    hd = W.shape[1]
    blk = 2000
    assert n % blk == 0
    grid = n // blk

    def body(x_ref, w_ref, b_ref, o_ref):
        o_ref[...] = (
            jnp.dot(x_ref[...], w_ref[...], preferred_element_type=jnp.float32)
            + b_ref[...]
        )

    return pl.pallas_call(
        body,
        grid=(grid,),
        in_specs=[
            pl.BlockSpec((blk, d_in), lambda i: (i, 0)),
            pl.BlockSpec((d_in, hd), lambda i: (0, 0)),
            pl.BlockSpec((1, hd), lambda i: (0, 0)),
        ],
        out_specs=pl.BlockSpec((blk, hd), lambda i: (i, 0)),
        out_shape=jax.ShapeDtypeStruct((n, hd), jnp.float32),
    )(x, W, b.reshape(1, hd))


def _tc_hedge_mlp(hs, W1s, b1, W2, b2, n_real):
    ph, hd = hs.shape
    blk = 2048
    assert ph % blk == 0
    grid = ph // blk

    def body(hs_ref, w1_ref, b1_ref, w2_ref, b2_ref, o_ref):
        i = pl.program_id(0)
        t = jnp.dot(hs_ref[...], w1_ref[...], preferred_element_type=jnp.float32)
        t = jnp.maximum(t + b1_ref[...], 0.0)
        o = jnp.dot(t, w2_ref[...], preferred_element_type=jnp.float32) + b2_ref[...]
        rows = i * blk + lax.broadcasted_iota(jnp.int32, (blk, 1), 0)
        o_ref[...] = jnp.where(rows < n_real, o, 0.0)

    return pl.pallas_call(
        body,
        grid=(grid,),
        in_specs=[
            pl.BlockSpec((blk, hd), lambda i: (i, 0)),
            pl.BlockSpec((hd, hd), lambda i: (0, 0)),
            pl.BlockSpec((1, hd), lambda i: (0, 0)),
            pl.BlockSpec((hd, hd), lambda i: (0, 0)),
            pl.BlockSpec((1, hd), lambda i: (0, 0)),
        ],
        out_specs=pl.BlockSpec((blk, hd), lambda i: (i, 0)),
        out_shape=jax.ShapeDtypeStruct((ph, hd), jnp.float32),
    )(hs, W1s, b1.reshape(1, hd), W2, b2.reshape(1, hd))


def _sc_gather_sum(x_t, idx_g, ph, hd, card):
    hpt = ph // NW
    shot_h = 32
    shots = hpt // shot_h
    idx_rows_shot = shot_h * card // 128
    rows_shot = shot_h * card

    mesh = plsc.VectorSubcoreMesh(
        core_axis_name="c", subcore_axis_name="s",
        num_cores=NC, num_subcores=NS)

    n_nodes = x_t.shape[0]
    stage_rows = n_nodes // NS

    def body(xt_hbm, idx_hbm, out_hbm, idx_v, rows_v, res_v, xt_sp, sem0, sem1):
        cid = lax.axis_index("c")
        sid = lax.axis_index("s")
        wid = sid * NC + cid
        sems = (sem0, sem1)

        pltpu.sync_copy(xt_hbm.at[pl.ds(sid * stage_rows, stage_rows)],
                        xt_sp.at[pl.ds(sid * stage_rows, stage_rows)])
        plsc.subcore_barrier()

        def fire(s, b):
            row0 = wid * (shots * idx_rows_shot) + s * idx_rows_shot
            pltpu.sync_copy(idx_hbm.at[pl.ds(row0, idx_rows_shot)], idx_v.at[b])
            hs = []
            for j in range(idx_rows_shot):
                hs.append(pltpu.async_copy(
                    xt_sp.at[idx_v.at[b, j]],
                    rows_v.at[b, pl.ds(j * 128, 128)],
                    sems[b]))
            return hs

        def reduce_shot(s, b):
            def hbody(h, _):
                base = h * card
                a0 = rows_v[b, base, pl.ds(0, LANES)]
                a1 = rows_v[b, base, pl.ds(LANES, LANES)]
                for c in range(1, card):
                    a0 = a0 + rows_v[b, base + c, pl.ds(0, LANES)]
                    a1 = a1 + rows_v[b, base + c, pl.ds(LANES, LANES)]
                res_v[s * shot_h + h, pl.ds(0, LANES)] = a0
                res_v[s * shot_h + h, pl.ds(LANES, LANES)] = a1
                return 0
            lax.fori_loop(0, shot_h, hbody, 0)

        pending = {0: fire(0, 0)}
        for s in range(shots):
            b = s % 2
            if s + 1 < shots:
                pending[s + 1] = fire(s + 1, (s + 1) % 2)
            for h in pending.pop(s):
                h.wait()
            reduce_shot(s, b)
        pltpu.sync_copy(res_v, out_hbm.at[pl.ds(wid * hpt, hpt)])

    call = pl.kernel(
        body,
        out_type=jax.ShapeDtypeStruct((ph, hd), jnp.float32),
        mesh=mesh,
        scratch_types=[
            pltpu.VMEM((2, idx_rows_shot, 128), jnp.int32),
            pltpu.VMEM((2, rows_shot, hd), jnp.float32),
            pltpu.VMEM((hpt, hd), jnp.float32),
            pltpu.VMEM_SHARED((x_t.shape[0], hd), jnp.float32),
            pltpu.SemaphoreType.DMA,
            pltpu.SemaphoreType.DMA,
        ],
        compiler_params=pltpu.CompilerParams(use_tc_tiling_on_sc=False, needs_layout_passes=False),
    )
    return call(x_t, idx_g)


_CHUNK = 80


def _sc_scatter(hedge_feat, idx_s, n_nodes, ph, hd, card, n_real):
    hpt = ph // NW
    hchunks = hpt // _CHUNK
    nk = card * hchunks
    rows_pt = n_nodes // NS
    zrows = hpt
    ncp = ((n_nodes + 127) // 128) * 128
    vpc = _CHUNK // LANES

    mesh = plsc.VectorSubcoreMesh(
        core_axis_name="c", subcore_axis_name="s",
        num_cores=NC, num_subcores=NS)

    def body(feat_hbm, idx_hbm, acc_hbm, cnt_hbm,
             feat_v, idx_v, ones_v, hist_v, acc_sp, sem):
        zero16 = jnp.zeros((LANES,), jnp.float32)
        iota16 = lax.iota(jnp.int32, LANES)
        cid = lax.axis_index("c")
        sid = lax.axis_index("s")
        wid = sid * NC + cid

        def zrow(r, _):
            feat_v[r, pl.ds(0, LANES)] = zero16
            feat_v[r, pl.ds(LANES, LANES)] = zero16
            return 0
        lax.fori_loop(0, zrows, zrow, 0)
        base = sid * rows_pt
        pltpu.sync_copy(feat_v, acc_sp.at[pl.ds(base, zrows)])
        pltpu.sync_copy(feat_v.at[pl.ds(0, rows_pt - zrows)],
                        acc_sp.at[pl.ds(base + zrows, rows_pt - zrows)])

        def fill(i, _):
            rows = wid * hpt + i * LANES + iota16
            ones_v[pl.ds(i * LANES, LANES)] = jnp.where(rows < n_real, 1.0, 0.0)
            return 0
        lax.fori_loop(0, hpt // LANES, fill, 0)

        def zh(i, _):
            hist_v[pl.ds(i * LANES, LANES)] = zero16
            return 0
        lax.fori_loop(0, ncp // LANES, zh, 0)

        plsc.subcore_barrier()

        pltpu.sync_copy(feat_hbm.at[pl.ds(wid * hpt, hpt)], feat_v)
        pltpu.sync_copy(idx_hbm.at[wid], idx_v)

        handles = []
        for k in range(nk):
            hc = k % hchunks
            handles.append(pltpu.async_copy(
                feat_v.at[pl.ds(hc * _CHUNK, _CHUNK)],
                acc_sp.at[idx_v.at[k]],
                sem, add=True))
            if len(handles) >= 8:
                handles.pop(0).wait()

        def hist(i, _):
            k = i // vpc
            j = i - k * vpc
            hc = lax.rem(k, hchunks)
            vidx = idx_v[k, pl.ds(j * LANES, LANES)]
            val = ones_v[pl.ds(hc * _CHUNK + j * LANES, LANES)]
            plsc.addupdate_scatter(hist_v, [vidx], val)
            return 0
        lax.fori_loop(0, nk * vpc, hist, 0)

        for h in handles:
            h.wait()

        plsc.subcore_barrier()

        pltpu.sync_copy(acc_sp.at[pl.ds(base, rows_pt)],
                        acc_hbm.at[cid, pl.ds(base, rows_pt)])
        pltpu.sync_copy(hist_v, cnt_hbm.at[wid])

    call = pl.kernel(
        body,
        out_type=(
            jax.ShapeDtypeStruct((NC, n_nodes, hd), jnp.float32),
            jax.ShapeDtypeStruct((NW, ncp), jnp.float32),
        ),
        mesh=mesh,
        scratch_types=[
            pltpu.VMEM((hpt, hd), jnp.float32),
            pltpu.VMEM((nk, _CHUNK), jnp.int32),
            pltpu.VMEM((hpt,), jnp.float32),
            pltpu.VMEM((ncp,), jnp.float32),
            pltpu.VMEM_SHARED((n_nodes, hd), jnp.float32),
            pltpu.SemaphoreType.DMA,
        ],
        compiler_params=pltpu.CompilerParams(use_tc_tiling_on_sc=False, needs_layout_passes=False),
    )
    return call(hedge_feat, idx_s)


def _tc_final(x, acc, cnt, W_h2n, b_h2n, W_u1, b_u1, W_u2, b_u2, gamma, beta):
    n, d_in = x.shape
    hd = acc.shape[-1]
    d_out = W_h2n.shape[1]
    blk = 2000
    assert n % blk == 0
    grid = n // blk

    def body(x_ref, acc_ref, cnt_ref, wh_ref, bh_ref, w1_ref, b1_ref,
             w2_ref, b2_ref, g_ref, be_ref, o_ref):
        cnt = jnp.maximum(jnp.sum(cnt_ref[...], axis=1), 1.0)
        nf = (acc_ref[0] + acc_ref[1]) / cnt[:, None]
        nf = jnp.dot(nf, wh_ref[...], preferred_element_type=jnp.float32) + bh_ref[...]
        w1 = w1_ref[...]
        u = (jnp.dot(x_ref[...], w1[:d_in], preferred_element_type=jnp.float32)
             + jnp.dot(nf, w1[d_in:], preferred_element_type=jnp.float32)
             + b1_ref[...])
        u = jnp.maximum(u, 0.0)
        o = jnp.dot(u, w2_ref[...], preferred_element_type=jnp.float32) + b2_ref[...]
        mu = jnp.mean(o, axis=-1, keepdims=True)
        var = jnp.mean((o - mu) ** 2, axis=-1, keepdims=True)
        o_ref[...] = (o - mu) * lax.rsqrt(var + 1e-5) * g_ref[...] + be_ref[...]

    return pl.pallas_call(
        body,
        grid=(grid,),
        in_specs=[
            pl.BlockSpec((blk, d_in), lambda i: (i, 0)),
            pl.BlockSpec((NC, blk, hd), lambda i: (0, i, 0)),
            pl.BlockSpec((blk, NW), lambda i: (i, 0)),
            pl.BlockSpec((hd, d_out), lambda i: (0, 0)),
            pl.BlockSpec((1, d_out), lambda i: (0, 0)),
            pl.BlockSpec((d_in + d_out, d_out), lambda i: (0, 0)),
            pl.BlockSpec((1, d_out), lambda i: (0, 0)),
            pl.BlockSpec((d_out, d_out), lambda i: (0, 0)),
            pl.BlockSpec((1, d_out), lambda i: (0, 0)),
            pl.BlockSpec((1, d_out), lambda i: (0, 0)),
            pl.BlockSpec((1, d_out), lambda i: (0, 0)),
        ],
        out_specs=pl.BlockSpec((blk, d_out), lambda i: (i, 0)),
        out_shape=jax.ShapeDtypeStruct((n, d_out), jnp.float32),
    )(x, acc, cnt, W_h2n, b_h2n.reshape(1, d_out), W_u1,
      b_u1.reshape(1, d_out), W_u2, b_u2.reshape(1, d_out),
      gamma.reshape(1, d_out), beta.reshape(1, d_out))


def kernel(x, W_n2h, b_n2h, W_h1, b_h1, W_h2, b_h2, W_h2n, b_h2n,
           W_u1, b_u1, W_u2, b_u2, gamma, beta, hyperedge_index):
    n_nodes = x.shape[0]
    h_real, card = hyperedge_index.shape
    hd = W_n2h.shape[1]

    ph = ((h_real + NW * 320 - 1) // (NW * 320)) * (NW * 320)
    hpt = ph // NW

    idx_pad = jnp.zeros((ph, card), jnp.int32).at[:h_real].set(hyperedge_index)
    idx_g = idx_pad.reshape(ph * card // 128, 128)
    idx_s = (idx_pad.reshape(NW, hpt, card)
             .transpose(0, 2, 1)
             .reshape(NW, card * (hpt // _CHUNK), _CHUNK))

    x_t = _tc_node_transform(x, W_n2h, b_n2h)
    hedge_sums = _sc_gather_sum(x_t, idx_g, ph, hd, card)
    hedge_feat = _tc_hedge_mlp(hedge_sums, W_h1 / card, b_h1, W_h2, b_h2, h_real)
    acc, cnt = _sc_scatter(hedge_feat, idx_s, n_nodes, ph, hd, card, h_real)
    return _tc_final(x, acc, cnt[:, :n_nodes].T, W_h2n, b_h2n, W_u1, b_u1,
                     W_u2, b_u2, gamma, beta)


# speedup vs baseline: 14.2618x; 1.0082x over previous
import functools

import jax
import jax.numpy as jnp
from jax import lax
from jax.experimental import pallas as pl
from jax.experimental.pallas import tpu as pltpu
from jax.experimental.pallas import tpu_sc as plsc

NC = 2
NS = 16
NW = NC * NS
LANES = 16


def _tc_node_transform(x, W, b):
    n, d_in = x.shape
    hd = W.shape[1]
    blk = 2000
    assert n % blk == 0
    grid = n // blk

    def body(x_ref, w_ref, b_ref, o_ref):
        o_ref[...] = (
            jnp.dot(x_ref[...], w_ref[...], preferred_element_type=jnp.float32)
            + b_ref[...]
        )

    return pl.pallas_call(
        body,
        grid=(grid,),
        in_specs=[
            pl.BlockSpec((blk, d_in), lambda i: (i, 0)),
            pl.BlockSpec((d_in, hd), lambda i: (0, 0)),
            pl.BlockSpec((1, hd), lambda i: (0, 0)),
        ],
        out_specs=pl.BlockSpec((blk, hd), lambda i: (i, 0)),
        out_shape=jax.ShapeDtypeStruct((n, hd), jnp.float32),
    )(x, W, b.reshape(1, hd))


def _tc_hedge_mlp(hs, W1s, b1, W2, b2, n_real):
    ph, hd = hs.shape
    blk = 2048
    assert ph % blk == 0
    grid = ph // blk

    def body(hs_ref, w1_ref, b1_ref, w2_ref, b2_ref, o_ref):
        i = pl.program_id(0)
        t = jnp.dot(hs_ref[...], w1_ref[...], preferred_element_type=jnp.float32)
        t = jnp.maximum(t + b1_ref[...], 0.0)
        o = jnp.dot(t, w2_ref[...], preferred_element_type=jnp.float32) + b2_ref[...]
        rows = i * blk + lax.broadcasted_iota(jnp.int32, (blk, 1), 0)
        o_ref[...] = jnp.where(rows < n_real, o, 0.0)

    return pl.pallas_call(
        body,
        grid=(grid,),
        in_specs=[
            pl.BlockSpec((blk, hd), lambda i: (i, 0)),
            pl.BlockSpec((hd, hd), lambda i: (0, 0)),
            pl.BlockSpec((1, hd), lambda i: (0, 0)),
            pl.BlockSpec((hd, hd), lambda i: (0, 0)),
            pl.BlockSpec((1, hd), lambda i: (0, 0)),
        ],
        out_specs=pl.BlockSpec((blk, hd), lambda i: (i, 0)),
        out_shape=jax.ShapeDtypeStruct((ph, hd), jnp.float32),
    )(hs, W1s, b1.reshape(1, hd), W2, b2.reshape(1, hd))


def _sc_gather_sum(x_t, idx_g, ph, hd, card):
    hpt = ph // NW
    shot_h = 32
    shots = hpt // shot_h
    idx_rows_shot = shot_h * card // 128
    rows_shot = shot_h * card

    mesh = plsc.VectorSubcoreMesh(
        core_axis_name="c", subcore_axis_name="s",
        num_cores=NC, num_subcores=NS)

    n_nodes = x_t.shape[0]
    stage_rows = n_nodes // NS

    def body(xt_hbm, idx_hbm, out_hbm, idx_v, rows_v, res_v, xt_sp, sem0, sem1):
        cid = lax.axis_index("c")
        sid = lax.axis_index("s")
        wid = sid * NC + cid
        sems = (sem0, sem1)

        pltpu.sync_copy(xt_hbm.at[pl.ds(sid * stage_rows, stage_rows)],
                        xt_sp.at[pl.ds(sid * stage_rows, stage_rows)])
        plsc.subcore_barrier()

        def fire(s, b):
            row0 = wid * (shots * idx_rows_shot) + s * idx_rows_shot
            pltpu.sync_copy(idx_hbm.at[pl.ds(row0, idx_rows_shot)], idx_v.at[b])
            hs = []
            for j in range(idx_rows_shot):
                hs.append(pltpu.async_copy(
                    xt_sp.at[idx_v.at[b, j]],
                    rows_v.at[b, pl.ds(j * 128, 128)],
                    sems[b]))
            return hs

        def reduce_shot(s, b):
            def hbody(h, _):
                base = h * card
                a0 = rows_v[b, base, pl.ds(0, LANES)]
                a1 = rows_v[b, base, pl.ds(LANES, LANES)]
                for c in range(1, card):
                    a0 = a0 + rows_v[b, base + c, pl.ds(0, LANES)]
                    a1 = a1 + rows_v[b, base + c, pl.ds(LANES, LANES)]
                res_v[s * shot_h + h, pl.ds(0, LANES)] = a0
                res_v[s * shot_h + h, pl.ds(LANES, LANES)] = a1
                return 0
            lax.fori_loop(0, shot_h, hbody, 0)

        pending = {0: fire(0, 0)}
        for s in range(shots):
            b = s % 2
            if s + 1 < shots:
                pending[s + 1] = fire(s + 1, (s + 1) % 2)
            for h in pending.pop(s):
                h.wait()
            reduce_shot(s, b)
        pltpu.sync_copy(res_v, out_hbm.at[pl.ds(wid * hpt, hpt)])

    call = pl.kernel(
        body,
        out_type=jax.ShapeDtypeStruct((ph, hd), jnp.float32),
        mesh=mesh,
        scratch_types=[
            pltpu.VMEM((2, idx_rows_shot, 128), jnp.int32),
            pltpu.VMEM((2, rows_shot, hd), jnp.float32),
            pltpu.VMEM((hpt, hd), jnp.float32),
            pltpu.VMEM_SHARED((x_t.shape[0], hd), jnp.float32),
            pltpu.SemaphoreType.DMA,
            pltpu.SemaphoreType.DMA,
        ],
        compiler_params=pltpu.CompilerParams(use_tc_tiling_on_sc=False, needs_layout_passes=False),
    )
    return call(x_t, idx_g)


_CHUNK = 80


def _sc_scatter(hedge_feat, idx_s, n_nodes, ph, hd, card, n_real):
    hpt = ph // NW
    hchunks = hpt // _CHUNK
    nk = card * hchunks
    rows_pt = n_nodes // NS
    zrows = hpt
    ncp = ((n_nodes + 127) // 128) * 128
    vpc = _CHUNK // LANES

    mesh = plsc.VectorSubcoreMesh(
        core_axis_name="c", subcore_axis_name="s",
        num_cores=NC, num_subcores=NS)

    def body(feat_hbm, idx_hbm, acc_hbm, cnt_hbm,
             feat_v, idx_v, ones_v, hist_v, acc_sp, sem):
        zero16 = jnp.zeros((LANES,), jnp.float32)
        iota16 = lax.iota(jnp.int32, LANES)
        cid = lax.axis_index("c")
        sid = lax.axis_index("s")
        wid = sid * NC + cid

        def zrow(r, _):
            feat_v[r, pl.ds(0, LANES)] = zero16
            feat_v[r, pl.ds(LANES, LANES)] = zero16
            return 0
        lax.fori_loop(0, zrows, zrow, 0)
        base = sid * rows_pt
        pltpu.sync_copy(feat_v, acc_sp.at[pl.ds(base, zrows)])
        pltpu.sync_copy(feat_v.at[pl.ds(0, rows_pt - zrows)],
                        acc_sp.at[pl.ds(base + zrows, rows_pt - zrows)])

        def fill(i, _):
            rows = wid * hpt + i * LANES + iota16
            ones_v[pl.ds(i * LANES, LANES)] = jnp.where(rows < n_real, 1.0, 0.0)
            return 0
        lax.fori_loop(0, hpt // LANES, fill, 0)

        def zh(i, _):
            hist_v[pl.ds(i * LANES, LANES)] = zero16
            return 0
        lax.fori_loop(0, ncp // LANES, zh, 0)

        plsc.subcore_barrier()

        pltpu.sync_copy(feat_hbm.at[pl.ds(wid * hpt, hpt)], feat_v)
        pltpu.sync_copy(idx_hbm.at[wid], idx_v)

        handles = []
        for k in range(nk):
            hc = k % hchunks
            handles.append(pltpu.async_copy(
                feat_v.at[pl.ds(hc * _CHUNK, _CHUNK)],
                acc_sp.at[idx_v.at[k]],
                sem, add=True))

        def hist(i, _):
            k = i // vpc
            j = i - k * vpc
            hc = lax.rem(k, hchunks)
            vidx = idx_v[k, pl.ds(j * LANES, LANES)]
            val = ones_v[pl.ds(hc * _CHUNK + j * LANES, LANES)]
            plsc.addupdate_scatter(hist_v, [vidx], val)
            return 0
        lax.fori_loop(0, nk * vpc, hist, 0)

        for h in handles:
            h.wait()

        plsc.subcore_barrier()

        pltpu.sync_copy(acc_sp.at[pl.ds(base, rows_pt)],
                        acc_hbm.at[cid, pl.ds(base, rows_pt)])
        pltpu.sync_copy(hist_v, cnt_hbm.at[wid])

    call = pl.kernel(
        body,
        out_type=(
            jax.ShapeDtypeStruct((NC, n_nodes, hd), jnp.float32),
            jax.ShapeDtypeStruct((NW, ncp), jnp.float32),
        ),
        mesh=mesh,
        scratch_types=[
            pltpu.VMEM((hpt, hd), jnp.float32),
            pltpu.VMEM((nk, _CHUNK), jnp.int32),
            pltpu.VMEM((hpt,), jnp.float32),
            pltpu.VMEM((ncp,), jnp.float32),
            pltpu.VMEM_SHARED((n_nodes, hd), jnp.float32),
            pltpu.SemaphoreType.DMA,
        ],
        compiler_params=pltpu.CompilerParams(use_tc_tiling_on_sc=False, needs_layout_passes=False),
    )
    return call(hedge_feat, idx_s)


def _tc_final(x, acc, cnt, W_h2n, b_h2n, W_u1, b_u1, W_u2, b_u2, gamma, beta):
    n, d_in = x.shape
    hd = acc.shape[-1]
    d_out = W_h2n.shape[1]
    blk = 2000
    assert n % blk == 0
    grid = n // blk

    def body(x_ref, acc_ref, cnt_ref, wh_ref, bh_ref, w1_ref, b1_ref,
             w2_ref, b2_ref, g_ref, be_ref, o_ref):
        cnt = jnp.maximum(jnp.sum(cnt_ref[...], axis=1), 1.0)
        nf = (acc_ref[0] + acc_ref[1]) / cnt[:, None]
        nf = jnp.dot(nf, wh_ref[...], preferred_element_type=jnp.float32) + bh_ref[...]
        w1 = w1_ref[...]
        u = (jnp.dot(x_ref[...], w1[:d_in], preferred_element_type=jnp.float32)
             + jnp.dot(nf, w1[d_in:], preferred_element_type=jnp.float32)
             + b1_ref[...])
        u = jnp.maximum(u, 0.0)
        o = jnp.dot(u, w2_ref[...], preferred_element_type=jnp.float32) + b2_ref[...]
        mu = jnp.mean(o, axis=-1, keepdims=True)
        var = jnp.mean((o - mu) ** 2, axis=-1, keepdims=True)
        o_ref[...] = (o - mu) * lax.rsqrt(var + 1e-5) * g_ref[...] + be_ref[...]

    return pl.pallas_call(
        body,
        grid=(grid,),
        in_specs=[
            pl.BlockSpec((blk, d_in), lambda i: (i, 0)),
            pl.BlockSpec((NC, blk, hd), lambda i: (0, i, 0)),
            pl.BlockSpec((blk, NW), lambda i: (i, 0)),
            pl.BlockSpec((hd, d_out), lambda i: (0, 0)),
            pl.BlockSpec((1, d_out), lambda i: (0, 0)),
            pl.BlockSpec((d_in + d_out, d_out), lambda i: (0, 0)),
            pl.BlockSpec((1, d_out), lambda i: (0, 0)),
            pl.BlockSpec((d_out, d_out), lambda i: (0, 0)),
            pl.BlockSpec((1, d_out), lambda i: (0, 0)),
            pl.BlockSpec((1, d_out), lambda i: (0, 0)),
            pl.BlockSpec((1, d_out), lambda i: (0, 0)),
        ],
        out_specs=pl.BlockSpec((blk, d_out), lambda i: (i, 0)),
        out_shape=jax.ShapeDtypeStruct((n, d_out), jnp.float32),
    )(x, acc, cnt, W_h2n, b_h2n.reshape(1, d_out), W_u1,
      b_u1.reshape(1, d_out), W_u2, b_u2.reshape(1, d_out),
      gamma.reshape(1, d_out), beta.reshape(1, d_out))


def kernel(x, W_n2h, b_n2h, W_h1, b_h1, W_h2, b_h2, W_h2n, b_h2n,
           W_u1, b_u1, W_u2, b_u2, gamma, beta, hyperedge_index):
    n_nodes = x.shape[0]
    h_real, card = hyperedge_index.shape
    hd = W_n2h.shape[1]

    ph = ((h_real + NW * 320 - 1) // (NW * 320)) * (NW * 320)
    hpt = ph // NW

    idx_pad = jnp.zeros((ph, card), jnp.int32).at[:h_real].set(hyperedge_index)
    idx_g = idx_pad.reshape(ph * card // 128, 128)
    idx_s = (idx_pad.reshape(NW, hpt, card)
             .transpose(0, 2, 1)
             .reshape(NW, card * (hpt // _CHUNK), _CHUNK))

    x_t = _tc_node_transform(x, W_n2h, b_n2h)
    hedge_sums = _sc_gather_sum(x_t, idx_g, ph, hd, card)
    hedge_feat = _tc_hedge_mlp(hedge_sums, W_h1 / card, b_h1, W_h2, b_h2, h_real)
    acc, cnt = _sc_scatter(hedge_feat, idx_s, n_nodes, ph, hd, card, h_real)
    return _tc_final(x, acc, cnt[:, :n_nodes].T, W_h2n, b_h2n, W_u1, b_u1,
                     W_u2, b_u2, gamma, beta)


# speedup vs baseline: 14.7518x; 1.0344x over previous
import functools

import jax
import jax.numpy as jnp
from jax import lax
from jax.experimental import pallas as pl
from jax.experimental.pallas import tpu as pltpu
from jax.experimental.pallas import tpu_sc as plsc

NC = 2
NS = 16
NW = NC * NS
LANES = 16


def _tc_node_transform(x, W, b):
    n, d_in = x.shape
    hd = W.shape[1]
    blk = 2000
    assert n % blk == 0
    grid = n // blk

    def body(x_ref, w_ref, b_ref, o_ref):
        o_ref[...] = (
            jnp.dot(x_ref[...], w_ref[...], preferred_element_type=jnp.float32)
            + b_ref[...]
        )

    return pl.pallas_call(
        body,
        grid=(grid,),
        in_specs=[
            pl.BlockSpec((blk, d_in), lambda i: (i, 0)),
            pl.BlockSpec((d_in, hd), lambda i: (0, 0)),
            pl.BlockSpec((1, hd), lambda i: (0, 0)),
        ],
        out_specs=pl.BlockSpec((blk, hd), lambda i: (i, 0)),
        out_shape=jax.ShapeDtypeStruct((n, hd), jnp.float32),
    )(x, W, b.reshape(1, hd))


def _tc_hedge_mlp(hs, W1s, b1, W2, b2, n_real):
    ph, hd = hs.shape
    blk = 2048
    assert ph % blk == 0
    grid = ph // blk

    def body(hs_ref, w1_ref, b1_ref, w2_ref, b2_ref, o_ref):
        i = pl.program_id(0)
        t = jnp.dot(hs_ref[...], w1_ref[...], preferred_element_type=jnp.float32)
        t = jnp.maximum(t + b1_ref[...], 0.0)
        o = jnp.dot(t, w2_ref[...], preferred_element_type=jnp.float32) + b2_ref[...]
        rows = i * blk + lax.broadcasted_iota(jnp.int32, (blk, 1), 0)
        o_ref[...] = jnp.where(rows < n_real, o, 0.0)

    return pl.pallas_call(
        body,
        grid=(grid,),
        in_specs=[
            pl.BlockSpec((blk, hd), lambda i: (i, 0)),
            pl.BlockSpec((hd, hd), lambda i: (0, 0)),
            pl.BlockSpec((1, hd), lambda i: (0, 0)),
            pl.BlockSpec((hd, hd), lambda i: (0, 0)),
            pl.BlockSpec((1, hd), lambda i: (0, 0)),
        ],
        out_specs=pl.BlockSpec((blk, hd), lambda i: (i, 0)),
        out_shape=jax.ShapeDtypeStruct((ph, hd), jnp.float32),
    )(hs, W1s, b1.reshape(1, hd), W2, b2.reshape(1, hd))


def _sc_gather_sum(x_t, idx_g, ph, hd, card):
    hpt = ph // NW
    shot_h = 32
    shots = hpt // shot_h
    idx_rows_shot = shot_h * card // 128
    rows_shot = shot_h * card

    mesh = plsc.VectorSubcoreMesh(
        core_axis_name="c", subcore_axis_name="s",
        num_cores=NC, num_subcores=NS)

    n_nodes = x_t.shape[0]
    stage_rows = n_nodes // NS

    def body(xt_hbm, idx_hbm, out_hbm, idx_v, rows_v, res_v, xt_sp, sem0, sem1):
        cid = lax.axis_index("c")
        sid = lax.axis_index("s")
        wid = sid * NC + cid
        sems = (sem0, sem1)

        pltpu.sync_copy(xt_hbm.at[pl.ds(sid * stage_rows, stage_rows)],
                        xt_sp.at[pl.ds(sid * stage_rows, stage_rows)])
        plsc.subcore_barrier()

        def fire(s, b):
            row0 = wid * (shots * idx_rows_shot) + s * idx_rows_shot
            pltpu.sync_copy(idx_hbm.at[pl.ds(row0, idx_rows_shot)], idx_v.at[b])
            hs = []
            for j in range(idx_rows_shot):
                hs.append(pltpu.async_copy(
                    xt_sp.at[idx_v.at[b, j]],
                    rows_v.at[b, pl.ds(j * 128, 128)],
                    sems[b]))
            return hs

        def reduce_shot(s, b):
            def hbody(h, _):
                base = h * card
                a0 = rows_v[b, base, pl.ds(0, LANES)]
                a1 = rows_v[b, base, pl.ds(LANES, LANES)]
                for c in range(1, card):
                    a0 = a0 + rows_v[b, base + c, pl.ds(0, LANES)]
                    a1 = a1 + rows_v[b, base + c, pl.ds(LANES, LANES)]
                res_v[s * shot_h + h, pl.ds(0, LANES)] = a0
                res_v[s * shot_h + h, pl.ds(LANES, LANES)] = a1
                return 0
            lax.fori_loop(0, shot_h, hbody, 0)

        pending = {0: fire(0, 0)}
        for s in range(shots):
            b = s % 2
            if s + 1 < shots:
                pending[s + 1] = fire(s + 1, (s + 1) % 2)
            for h in pending.pop(s):
                h.wait()
            reduce_shot(s, b)
        pltpu.sync_copy(res_v, out_hbm.at[pl.ds(wid * hpt, hpt)])

    call = pl.kernel(
        body,
        out_type=jax.ShapeDtypeStruct((ph, hd), jnp.float32),
        mesh=mesh,
        scratch_types=[
            pltpu.VMEM((2, idx_rows_shot, 128), jnp.int32),
            pltpu.VMEM((2, rows_shot, hd), jnp.float32),
            pltpu.VMEM((hpt, hd), jnp.float32),
            pltpu.VMEM_SHARED((x_t.shape[0], hd), jnp.float32),
            pltpu.SemaphoreType.DMA,
            pltpu.SemaphoreType.DMA,
        ],
        compiler_params=pltpu.CompilerParams(use_tc_tiling_on_sc=False, needs_layout_passes=False),
    )
    return call(x_t, idx_g)


_CHUNK = 80


def _sc_scatter(hedge_feat, idx_s, n_nodes, ph, hd, card, n_real):
    hpt = ph // NW
    hchunks = hpt // _CHUNK
    nk = card * hchunks
    rows_pt = n_nodes // NS
    zrows = hpt
    ncp = NS * ((n_nodes + NS * 8 - 1) // (NS * 8)) * 8
    cnt_pt = ncp // NS

    mesh = plsc.VectorSubcoreMesh(
        core_axis_name="c", subcore_axis_name="s",
        num_cores=NC, num_subcores=NS)

    def body(feat_hbm, idx_hbm, acc_hbm, cnt_hbm,
             feat_v, idx_v, ones_v, zc_v, acc_sp, cnt_sp, sem):
        zero16 = jnp.zeros((LANES,), jnp.float32)
        iota16 = lax.iota(jnp.int32, LANES)
        cid = lax.axis_index("c")
        sid = lax.axis_index("s")
        wid = sid * NC + cid

        def zrow(r, _):
            feat_v[r, pl.ds(0, LANES)] = zero16
            feat_v[r, pl.ds(LANES, LANES)] = zero16
            return 0
        lax.fori_loop(0, zrows, zrow, 0)
        base = sid * rows_pt
        pltpu.sync_copy(feat_v, acc_sp.at[pl.ds(base, zrows)])
        pltpu.sync_copy(feat_v.at[pl.ds(0, rows_pt - zrows)],
                        acc_sp.at[pl.ds(base + zrows, rows_pt - zrows)])

        def fill(i, _):
            rows = wid * hpt + i * LANES + iota16
            ones_v[pl.ds(i * LANES, LANES)] = jnp.where(rows < n_real, 1.0, 0.0)
            return 0
        lax.fori_loop(0, hpt // LANES, fill, 0)

        def zcnt(i, _):
            zc_v[pl.ds(i * LANES, LANES)] = zero16
            return 0
        lax.fori_loop(0, cnt_pt // LANES, zcnt, 0)
        pltpu.sync_copy(zc_v, cnt_sp.at[pl.ds(sid * cnt_pt, cnt_pt)])

        plsc.subcore_barrier()

        pltpu.sync_copy(feat_hbm.at[pl.ds(wid * hpt, hpt)], feat_v)
        pltpu.sync_copy(idx_hbm.at[wid], idx_v)

        handles = []
        for k in range(nk):
            hc = k % hchunks
            handles.append(pltpu.async_copy(
                feat_v.at[pl.ds(hc * _CHUNK, _CHUNK)],
                acc_sp.at[idx_v.at[k]],
                sem, add=True))
            handles.append(pltpu.async_copy(
                ones_v.at[pl.ds(hc * _CHUNK, _CHUNK)],
                cnt_sp.at[idx_v.at[k]],
                sem, add=True))
        for h in handles:
            h.wait()

        plsc.subcore_barrier()

        pltpu.sync_copy(acc_sp.at[pl.ds(base, rows_pt)],
                        acc_hbm.at[cid, pl.ds(base, rows_pt)])
        pltpu.sync_copy(cnt_sp.at[pl.ds(sid * cnt_pt, cnt_pt)],
                        cnt_hbm.at[cid, pl.ds(sid * cnt_pt, cnt_pt)])

    call = pl.kernel(
        body,
        out_type=(
            jax.ShapeDtypeStruct((NC, n_nodes, hd), jnp.float32),
            jax.ShapeDtypeStruct((NC, ncp), jnp.float32),
        ),
        mesh=mesh,
        scratch_types=[
            pltpu.VMEM((hpt, hd), jnp.float32),
            pltpu.VMEM((nk, _CHUNK), jnp.int32),
            pltpu.VMEM((hpt,), jnp.float32),
            pltpu.VMEM((cnt_pt,), jnp.float32),
            pltpu.VMEM_SHARED((n_nodes, hd), jnp.float32),
            pltpu.VMEM_SHARED((ncp,), jnp.float32),
            pltpu.SemaphoreType.DMA,
        ],
        compiler_params=pltpu.CompilerParams(use_tc_tiling_on_sc=False, needs_layout_passes=False),
    )
    return call(hedge_feat, idx_s)


def _tc_final(x, acc, cnt, W_h2n, b_h2n, W_u1, b_u1, W_u2, b_u2, gamma, beta):
    n, d_in = x.shape
    hd = acc.shape[-1]
    d_out = W_h2n.shape[1]
    blk = 2000
    assert n % blk == 0
    grid = n // blk

    def body(x_ref, acc_ref, cnt_ref, wh_ref, bh_ref, w1_ref, b1_ref,
             w2_ref, b2_ref, g_ref, be_ref, o_ref):
        cnt = jnp.maximum(jnp.sum(cnt_ref[...], axis=1), 1.0)
        nf = (acc_ref[0] + acc_ref[1]) / cnt[:, None]
        nf = jnp.dot(nf, wh_ref[...], preferred_element_type=jnp.float32) + bh_ref[...]
        w1 = w1_ref[...]
        u = (jnp.dot(x_ref[...], w1[:d_in], preferred_element_type=jnp.float32)
             + jnp.dot(nf, w1[d_in:], preferred_element_type=jnp.float32)
             + b1_ref[...])
        u = jnp.maximum(u, 0.0)
        o = jnp.dot(u, w2_ref[...], preferred_element_type=jnp.float32) + b2_ref[...]
        mu = jnp.mean(o, axis=-1, keepdims=True)
        var = jnp.mean((o - mu) ** 2, axis=-1, keepdims=True)
        o_ref[...] = (o - mu) * lax.rsqrt(var + 1e-5) * g_ref[...] + be_ref[...]

    return pl.pallas_call(
        body,
        grid=(grid,),
        in_specs=[
            pl.BlockSpec((blk, d_in), lambda i: (i, 0)),
            pl.BlockSpec((NC, blk, hd), lambda i: (0, i, 0)),
            pl.BlockSpec((blk, NC), lambda i: (i, 0)),
            pl.BlockSpec((hd, d_out), lambda i: (0, 0)),
            pl.BlockSpec((1, d_out), lambda i: (0, 0)),
            pl.BlockSpec((d_in + d_out, d_out), lambda i: (0, 0)),
            pl.BlockSpec((1, d_out), lambda i: (0, 0)),
            pl.BlockSpec((d_out, d_out), lambda i: (0, 0)),
            pl.BlockSpec((1, d_out), lambda i: (0, 0)),
            pl.BlockSpec((1, d_out), lambda i: (0, 0)),
            pl.BlockSpec((1, d_out), lambda i: (0, 0)),
        ],
        out_specs=pl.BlockSpec((blk, d_out), lambda i: (i, 0)),
        out_shape=jax.ShapeDtypeStruct((n, d_out), jnp.float32),
    )(x, acc, cnt, W_h2n, b_h2n.reshape(1, d_out), W_u1,
      b_u1.reshape(1, d_out), W_u2, b_u2.reshape(1, d_out),
      gamma.reshape(1, d_out), beta.reshape(1, d_out))


def kernel(x, W_n2h, b_n2h, W_h1, b_h1, W_h2, b_h2, W_h2n, b_h2n,
           W_u1, b_u1, W_u2, b_u2, gamma, beta, hyperedge_index):
    n_nodes = x.shape[0]
    h_real, card = hyperedge_index.shape
    hd = W_n2h.shape[1]

    ph = ((h_real + NW * 320 - 1) // (NW * 320)) * (NW * 320)
    hpt = ph // NW

    idx_pad = jnp.zeros((ph, card), jnp.int32).at[:h_real].set(hyperedge_index)
    idx_g = idx_pad.reshape(ph * card // 128, 128)
    idx_s = (idx_pad.reshape(NW, hpt, card)
             .transpose(0, 2, 1)
             .reshape(NW, card * (hpt // _CHUNK), _CHUNK))

    x_t = _tc_node_transform(x, W_n2h, b_n2h)
    hedge_sums = _sc_gather_sum(x_t, idx_g, ph, hd, card)
    hedge_feat = _tc_hedge_mlp(hedge_sums, W_h1 / card, b_h1, W_h2, b_h2, h_real)
    acc, cnt = _sc_scatter(hedge_feat, idx_s, n_nodes, ph, hd, card, h_real)
    return _tc_final(x, acc, cnt[:, :n_nodes].T, W_h2n, b_h2n, W_u1, b_u1,
                     W_u2, b_u2, gamma, beta)


# speedup vs baseline: 14.8344x; 1.0056x over previous
import functools

import jax
import jax.numpy as jnp
from jax import lax
from jax.experimental import pallas as pl
from jax.experimental.pallas import tpu as pltpu
from jax.experimental.pallas import tpu_sc as plsc

NC = 2
NS = 16
NW = NC * NS
LANES = 16


def _tc_node_transform(x, W, b):
    n, d_in = x.shape
    hd = W.shape[1]
    blk = 2000
    assert n % blk == 0
    grid = n // blk

    def body(x_ref, w_ref, b_ref, o_ref):
        o_ref[...] = (
            jnp.dot(x_ref[...], w_ref[...], preferred_element_type=jnp.float32)
            + b_ref[...]
        )

    return pl.pallas_call(
        body,
        grid=(grid,),
        in_specs=[
            pl.BlockSpec((blk, d_in), lambda i: (i, 0)),
            pl.BlockSpec((d_in, hd), lambda i: (0, 0)),
            pl.BlockSpec((1, hd), lambda i: (0, 0)),
        ],
        out_specs=pl.BlockSpec((blk, hd), lambda i: (i, 0)),
        out_shape=jax.ShapeDtypeStruct((n, hd), jnp.float32),
    )(x, W, b.reshape(1, hd))


def _tc_hedge_mlp(hs, W1s, b1, W2, b2, n_real):
    ph, hd = hs.shape
    blk = 2048
    assert ph % blk == 0
    grid = ph // blk

    def body(hs_ref, w1_ref, b1_ref, w2_ref, b2_ref, o_ref):
        i = pl.program_id(0)
        t = jnp.dot(hs_ref[...], w1_ref[...], preferred_element_type=jnp.float32)
        t = jnp.maximum(t + b1_ref[...], 0.0)
        o = jnp.dot(t, w2_ref[...], preferred_element_type=jnp.float32) + b2_ref[...]
        rows = i * blk + lax.broadcasted_iota(jnp.int32, (blk, 1), 0)
        o_ref[...] = jnp.where(rows < n_real, o, 0.0)

    return pl.pallas_call(
        body,
        grid=(grid,),
        in_specs=[
            pl.BlockSpec((blk, hd), lambda i: (i, 0)),
            pl.BlockSpec((hd, hd), lambda i: (0, 0)),
            pl.BlockSpec((1, hd), lambda i: (0, 0)),
            pl.BlockSpec((hd, hd), lambda i: (0, 0)),
            pl.BlockSpec((1, hd), lambda i: (0, 0)),
        ],
        out_specs=pl.BlockSpec((blk, hd), lambda i: (i, 0)),
        out_shape=jax.ShapeDtypeStruct((ph, hd), jnp.float32),
    )(hs, W1s, b1.reshape(1, hd), W2, b2.reshape(1, hd))


def _sc_gather_sum(x_t, idx_g, ph, hd, card):
    hpt = ph // NW
    shot_h = 32
    shots = hpt // shot_h
    idx_rows_shot = shot_h * card // 128
    rows_shot = shot_h * card

    mesh = plsc.VectorSubcoreMesh(
        core_axis_name="c", subcore_axis_name="s",
        num_cores=NC, num_subcores=NS)

    n_nodes = x_t.shape[0]
    stage_rows = n_nodes // NS

    def body(xt_hbm, idx_hbm, out_hbm, idx_v, rows_v, res_v, xt_sp, sem0, sem1):
        cid = lax.axis_index("c")
        sid = lax.axis_index("s")
        wid = sid * NC + cid
        sems = (sem0, sem1)

        pltpu.sync_copy(xt_hbm.at[pl.ds(sid * stage_rows, stage_rows)],
                        xt_sp.at[pl.ds(sid * stage_rows, stage_rows)])
        plsc.subcore_barrier()

        def fire(s, b):
            row0 = wid * (shots * idx_rows_shot) + s * idx_rows_shot
            pltpu.sync_copy(idx_hbm.at[pl.ds(row0, idx_rows_shot)], idx_v.at[b])
            hs = []
            for j in range(idx_rows_shot):
                hs.append(pltpu.async_copy(
                    xt_sp.at[idx_v.at[b, j]],
                    rows_v.at[b, pl.ds(j * 128, 128)],
                    sems[b]))
            return hs

        def reduce_shot(s, b):
            def hbody(h, _):
                base = h * card
                a0 = rows_v[b, base, pl.ds(0, LANES)]
                a1 = rows_v[b, base, pl.ds(LANES, LANES)]
                for c in range(1, card):
                    a0 = a0 + rows_v[b, base + c, pl.ds(0, LANES)]
                    a1 = a1 + rows_v[b, base + c, pl.ds(LANES, LANES)]
                res_v[s * shot_h + h, pl.ds(0, LANES)] = a0
                res_v[s * shot_h + h, pl.ds(LANES, LANES)] = a1
                return 0
            lax.fori_loop(0, shot_h, hbody, 0)

        pending = {0: fire(0, 0)}
        for s in range(shots):
            b = s % 2
            if s + 1 < shots:
                pending[s + 1] = fire(s + 1, (s + 1) % 2)
            for h in pending.pop(s):
                h.wait()
            reduce_shot(s, b)
        pltpu.sync_copy(res_v, out_hbm.at[pl.ds(wid * hpt, hpt)])

    call = pl.kernel(
        body,
        out_type=jax.ShapeDtypeStruct((ph, hd), jnp.float32),
        mesh=mesh,
        scratch_types=[
            pltpu.VMEM((2, idx_rows_shot, 128), jnp.int32),
            pltpu.VMEM((2, rows_shot, hd), jnp.float32),
            pltpu.VMEM((hpt, hd), jnp.float32),
            pltpu.VMEM_SHARED((x_t.shape[0], hd), jnp.float32),
            pltpu.SemaphoreType.DMA,
            pltpu.SemaphoreType.DMA,
        ],
        compiler_params=pltpu.CompilerParams(use_tc_tiling_on_sc=False, needs_layout_passes=False),
    )
    return call(x_t, idx_g)


_CHUNK = 64


def _sc_scatter(hedge_feat, idx_s, n_nodes, ph, hd, card, n_real):
    hpt = ph // NW
    hchunks = hpt // _CHUNK
    nk = card * hchunks
    rows_pt = n_nodes // NS
    zrows = hpt
    ncp = NS * ((n_nodes + NS * 8 - 1) // (NS * 8)) * 8
    cnt_pt = ncp // NS

    mesh = plsc.VectorSubcoreMesh(
        core_axis_name="c", subcore_axis_name="s",
        num_cores=NC, num_subcores=NS)

    def body(feat_hbm, idx_hbm, acc_hbm, cnt_hbm,
             feat_v, idx_v, ones_v, zc_v, acc_sp, cnt_sp, sem):
        zero16 = jnp.zeros((LANES,), jnp.float32)
        iota16 = lax.iota(jnp.int32, LANES)
        cid = lax.axis_index("c")
        sid = lax.axis_index("s")
        wid = sid * NC + cid

        def zrow(r, _):
            feat_v[r, pl.ds(0, LANES)] = zero16
            feat_v[r, pl.ds(LANES, LANES)] = zero16
            return 0
        lax.fori_loop(0, zrows, zrow, 0)
        base = sid * rows_pt
        pltpu.sync_copy(feat_v, acc_sp.at[pl.ds(base, zrows)])
        pltpu.sync_copy(feat_v.at[pl.ds(0, rows_pt - zrows)],
                        acc_sp.at[pl.ds(base + zrows, rows_pt - zrows)])

        def fill(i, _):
            rows = wid * hpt + i * LANES + iota16
            ones_v[pl.ds(i * LANES, LANES)] = jnp.where(rows < n_real, 1.0, 0.0)
            return 0
        lax.fori_loop(0, hpt // LANES, fill, 0)

        def zcnt(i, _):
            zc_v[pl.ds(i * LANES, LANES)] = zero16
            return 0
        lax.fori_loop(0, cnt_pt // LANES, zcnt, 0)
        pltpu.sync_copy(zc_v, cnt_sp.at[pl.ds(sid * cnt_pt, cnt_pt)])

        plsc.subcore_barrier()

        pltpu.sync_copy(feat_hbm.at[pl.ds(wid * hpt, hpt)], feat_v)
        pltpu.sync_copy(idx_hbm.at[wid], idx_v)

        handles = []
        for k in range(nk):
            hc = k % hchunks
            handles.append(pltpu.async_copy(
                feat_v.at[pl.ds(hc * _CHUNK, _CHUNK)],
                acc_sp.at[idx_v.at[k]],
                sem, add=True))
            handles.append(pltpu.async_copy(
                ones_v.at[pl.ds(hc * _CHUNK, _CHUNK)],
                cnt_sp.at[idx_v.at[k]],
                sem, add=True))
            if len(handles) >= 8:
                for h in handles:
                    h.wait()
                handles = []
        for h in handles:
            h.wait()

        plsc.subcore_barrier()

        pltpu.sync_copy(acc_sp.at[pl.ds(base, rows_pt)],
                        acc_hbm.at[cid, pl.ds(base, rows_pt)])
        pltpu.sync_copy(cnt_sp.at[pl.ds(sid * cnt_pt, cnt_pt)],
                        cnt_hbm.at[cid, pl.ds(sid * cnt_pt, cnt_pt)])

    call = pl.kernel(
        body,
        out_type=(
            jax.ShapeDtypeStruct((NC, n_nodes, hd), jnp.float32),
            jax.ShapeDtypeStruct((NC, ncp), jnp.float32),
        ),
        mesh=mesh,
        scratch_types=[
            pltpu.VMEM((hpt, hd), jnp.float32),
            pltpu.VMEM((nk, _CHUNK), jnp.int32),
            pltpu.VMEM((hpt,), jnp.float32),
            pltpu.VMEM((cnt_pt,), jnp.float32),
            pltpu.VMEM_SHARED((n_nodes, hd), jnp.float32),
            pltpu.VMEM_SHARED((ncp,), jnp.float32),
            pltpu.SemaphoreType.DMA,
        ],
        compiler_params=pltpu.CompilerParams(use_tc_tiling_on_sc=False, needs_layout_passes=False),
    )
    return call(hedge_feat, idx_s)


def _tc_final(x, acc, cnt, W_h2n, b_h2n, W_u1, b_u1, W_u2, b_u2, gamma, beta):
    n, d_in = x.shape
    hd = acc.shape[-1]
    d_out = W_h2n.shape[1]
    blk = 2000
    assert n % blk == 0
    grid = n // blk

    def body(x_ref, acc_ref, cnt_ref, wh_ref, bh_ref, w1_ref, b1_ref,
             w2_ref, b2_ref, g_ref, be_ref, o_ref):
        cnt = jnp.maximum(jnp.sum(cnt_ref[...], axis=1), 1.0)
        nf = (acc_ref[0] + acc_ref[1]) / cnt[:, None]
        nf = jnp.dot(nf, wh_ref[...], preferred_element_type=jnp.float32) + bh_ref[...]
        w1 = w1_ref[...]
        u = (jnp.dot(x_ref[...], w1[:d_in], preferred_element_type=jnp.float32)
             + jnp.dot(nf, w1[d_in:], preferred_element_type=jnp.float32)
             + b1_ref[...])
        u = jnp.maximum(u, 0.0)
        o = jnp.dot(u, w2_ref[...], preferred_element_type=jnp.float32) + b2_ref[...]
        mu = jnp.mean(o, axis=-1, keepdims=True)
        var = jnp.mean((o - mu) ** 2, axis=-1, keepdims=True)
        o_ref[...] = (o - mu) * lax.rsqrt(var + 1e-5) * g_ref[...] + be_ref[...]

    return pl.pallas_call(
        body,
        grid=(grid,),
        in_specs=[
            pl.BlockSpec((blk, d_in), lambda i: (i, 0)),
            pl.BlockSpec((NC, blk, hd), lambda i: (0, i, 0)),
            pl.BlockSpec((blk, NC), lambda i: (i, 0)),
            pl.BlockSpec((hd, d_out), lambda i: (0, 0)),
            pl.BlockSpec((1, d_out), lambda i: (0, 0)),
            pl.BlockSpec((d_in + d_out, d_out), lambda i: (0, 0)),
            pl.BlockSpec((1, d_out), lambda i: (0, 0)),
            pl.BlockSpec((d_out, d_out), lambda i: (0, 0)),
            pl.BlockSpec((1, d_out), lambda i: (0, 0)),
            pl.BlockSpec((1, d_out), lambda i: (0, 0)),
            pl.BlockSpec((1, d_out), lambda i: (0, 0)),
        ],
        out_specs=pl.BlockSpec((blk, d_out), lambda i: (i, 0)),
        out_shape=jax.ShapeDtypeStruct((n, d_out), jnp.float32),
    )(x, acc, cnt, W_h2n, b_h2n.reshape(1, d_out), W_u1,
      b_u1.reshape(1, d_out), W_u2, b_u2.reshape(1, d_out),
      gamma.reshape(1, d_out), beta.reshape(1, d_out))


def kernel(x, W_n2h, b_n2h, W_h1, b_h1, W_h2, b_h2, W_h2n, b_h2n,
           W_u1, b_u1, W_u2, b_u2, gamma, beta, hyperedge_index):
    n_nodes = x.shape[0]
    h_real, card = hyperedge_index.shape
    hd = W_n2h.shape[1]

    ph = ((h_real + NW * 320 - 1) // (NW * 320)) * (NW * 320)
    hpt = ph // NW

    idx_pad = jnp.zeros((ph, card), jnp.int32).at[:h_real].set(hyperedge_index)
    idx_g = idx_pad.reshape(ph * card // 128, 128)
    idx_s = (idx_pad.reshape(NW, hpt, card)
             .transpose(0, 2, 1)
             .reshape(NW, card * (hpt // _CHUNK), _CHUNK))

    x_t = _tc_node_transform(x, W_n2h, b_n2h)
    hedge_sums = _sc_gather_sum(x_t, idx_g, ph, hd, card)
    hedge_feat = _tc_hedge_mlp(hedge_sums, W_h1 / card, b_h1, W_h2, b_h2, h_real)
    acc, cnt = _sc_scatter(hedge_feat, idx_s, n_nodes, ph, hd, card, h_real)
    return _tc_final(x, acc, cnt[:, :n_nodes].T, W_h2n, b_h2n, W_u1, b_u1,
                     W_u2, b_u2, gamma, beta)
